# baseline (speedup 1.0000x reference)
import functools

import jax
import jax.numpy as jnp
from jax import lax
from jax.experimental import pallas as pl
from jax.experimental.pallas import tpu as pltpu
from jax.experimental.pallas import tpu_sc as plsc

_MAX_LEN = 8192
_D_MODEL = 768
_NUM_WORKERS = 32
_ROWS_PER_WORKER = _MAX_LEN // _NUM_WORKERS


@functools.partial(
    pl.kernel,
    out_type=jax.ShapeDtypeStruct((_MAX_LEN, _D_MODEL), jnp.float32),
    mesh=plsc.VectorSubcoreMesh(core_axis_name="c", subcore_axis_name="s"),
)
def _pos_encoding_copy(emb_hbm, out_hbm):
    wid = lax.axis_index("s") * 2 + lax.axis_index("c")
    base = wid * _ROWS_PER_WORKER
    pltpu.sync_copy(
        emb_hbm.at[pl.ds(base, _ROWS_PER_WORKER)],
        out_hbm.at[pl.ds(base, _ROWS_PER_WORKER)],
    )


def kernel(x, embedding):
    del x
    return _pos_encoding_copy(embedding)[None]


# speedup vs baseline: 1.0036x; 1.0036x over previous
import functools

import jax
import jax.numpy as jnp
from jax import lax
from jax.experimental import pallas as pl
from jax.experimental.pallas import tpu as pltpu
from jax.experimental.pallas import tpu_sc as plsc

_MAX_LEN = 8192
_D_MODEL = 768
_NUM_WORKERS = 32
_ROWS_PER_WORKER = _MAX_LEN // _NUM_WORKERS


_CHUNKS_PER_WORKER = 8
_CHUNK_ROWS = _ROWS_PER_WORKER // _CHUNKS_PER_WORKER


@functools.partial(
    pl.kernel,
    out_type=jax.ShapeDtypeStruct((_MAX_LEN, _D_MODEL), jnp.float32),
    mesh=plsc.VectorSubcoreMesh(core_axis_name="c", subcore_axis_name="s"),
    scratch_types=[pltpu.SemaphoreType.DMA],
)
def _pos_encoding_copy(emb_hbm, out_hbm, sem):
    wid = lax.axis_index("s") * 2 + lax.axis_index("c")
    base = wid * _ROWS_PER_WORKER
    copies = []
    for j in range(_CHUNKS_PER_WORKER):
        lo = base + j * _CHUNK_ROWS
        copies.append(
            pltpu.async_copy(
                emb_hbm.at[pl.ds(lo, _CHUNK_ROWS)],
                out_hbm.at[pl.ds(lo, _CHUNK_ROWS)],
                sem,
            )
        )
    for c in copies:
        c.wait()


def kernel(x, embedding):
    del x
    return _pos_encoding_copy(embedding)[None]


# speedup vs baseline: 21.6810x; 21.6032x over previous
import functools

import jax
import jax.numpy as jnp
from jax import lax
from jax.experimental import pallas as pl
from jax.experimental.pallas import tpu as pltpu
from jax.experimental.pallas import tpu_sc as plsc

_MAX_LEN = 8192
_D_MODEL = 768
_NUM_WORKERS = 32
_ROWS_PER_WORKER = _MAX_LEN // _NUM_WORKERS


_CHUNK_ROWS = 64
_NUM_CHUNKS = _ROWS_PER_WORKER // _CHUNK_ROWS
_NBUF = 2


@functools.partial(
    pl.kernel,
    out_type=jax.ShapeDtypeStruct((_MAX_LEN, _D_MODEL), jnp.float32),
    mesh=plsc.VectorSubcoreMesh(core_axis_name="c", subcore_axis_name="s"),
    scratch_types=[
        pltpu.VMEM((_NBUF, _CHUNK_ROWS, _D_MODEL), jnp.float32),
        pltpu.SemaphoreType.DMA,
        pltpu.SemaphoreType.DMA,
    ],
)
def _pos_encoding_copy(emb_hbm, out_hbm, buf, in_sem, out_sem):
    wid = lax.axis_index("s") * 2 + lax.axis_index("c")
    base = wid * _ROWS_PER_WORKER

    def copy_in(j):
        return pltpu.async_copy(
            emb_hbm.at[pl.ds(base + j * _CHUNK_ROWS, _CHUNK_ROWS)],
            buf.at[j % _NBUF],
            in_sem,
        )

    def copy_out(j):
        return pltpu.async_copy(
            buf.at[j % _NBUF],
            out_hbm.at[pl.ds(base + j * _CHUNK_ROWS, _CHUNK_ROWS)],
            out_sem,
        )

    ins = [None] * _NUM_CHUNKS
    outs = [None] * _NUM_CHUNKS
    for j in range(_NBUF):
        ins[j] = copy_in(j)
    for j in range(_NUM_CHUNKS):
        ins[j].wait()
        outs[j] = copy_out(j)
        nxt = j + _NBUF
        if nxt < _NUM_CHUNKS:
            outs[j].wait()
            ins[nxt] = copy_in(nxt)
    for j in range(_NUM_CHUNKS - _NBUF, _NUM_CHUNKS):
        if j >= 0:
            outs[j].wait()


def kernel(x, embedding):
    del x
    return _pos_encoding_copy(embedding)[None]
